# baseline (speedup 1.0000x reference)
import functools

import jax
import jax.numpy as jnp
from jax import lax
from jax.experimental import pallas as pl
from jax.experimental.pallas import tpu as pltpu
from jax.experimental.pallas import tpu_sc as plsc

_MIN_NORM = 1e-15
_LANES = 16
_NC = 2
_NS = 16
_EB = 128


def _artanh(z):
    z = jnp.clip(z, -1.0 + 1e-7, 1.0 - 1e-7)
    return 0.5 * (jnp.log1p(z) - jnp.log1p(-z))


def _rnorm(v):
    return jnp.maximum(jnp.sqrt(jnp.sum(v * v, axis=-1, keepdims=True)),
                       _MIN_NORM)


def _proj(v):
    n = _rnorm(v)
    maxn = 1.0 - 1e-5
    return jnp.where(n > maxn, v / n * maxn, v)


def _expmap0(u):
    n = _rnorm(u)
    return jnp.tanh(n) * u / n


def _logmap0(p):
    n = _rnorm(p)
    return p / n * _artanh(n)


def _mobius_add(x, y):
    x2 = jnp.sum(x * x, axis=-1, keepdims=True)
    y2 = jnp.sum(y * y, axis=-1, keepdims=True)
    xy = jnp.sum(x * y, axis=-1, keepdims=True)
    num = (1.0 + 2.0 * xy + y2) * x + (1.0 - x2) * y
    den = 1.0 + 2.0 * xy + x2 * y2
    return num / jnp.maximum(den, _MIN_NORM)


def _linear_body(x_ref, w_ref, b_ref, o_ref):
    xb = x_ref[...]
    w = w_ref[...]
    b = b_ref[...]
    mx = lax.dot_general(xb, w, (((1,), (1,)), ((), ())),
                         preferred_element_type=jnp.float32)
    x_n = _rnorm(xb)
    mx_n = _rnorm(mx)
    mv = jnp.tanh(mx_n / x_n * _artanh(x_n)) * mx / mx_n
    res = _proj(mv)
    hb = _proj(_expmap0(b))
    res = _proj(_mobius_add(res, hb))
    o_ref[...] = _logmap0(res)


def _epilogue_body(a_ref, b_ref, o_ref):
    s = a_ref[...] + b_ref[...]
    agg = _proj(_expmap0(s))
    xt = jnp.maximum(_logmap0(agg), 0.0)
    o_ref[...] = _proj(_expmap0(xt))


def _make_agg(n_nodes, d, e_per_tile):
    nb = e_per_tile // _EB
    assert nb % 2 == 0
    nb_c = 8
    if nb % 16 == 0:
        nb_c = 16
    assert nb % nb_c == 0
    n_chunks = nb // nb_c
    nsb_c = nb_c // 2
    sub_stride = 624
    n_wr = 5
    wr = _EB
    assert (_NS - 1) * sub_stride + n_wr * wr == n_nodes
    mesh = plsc.VectorSubcoreMesh(core_axis_name="c", subcore_axis_name="s")

    @functools.partial(
        pl.kernel,
        mesh=mesh,
        out_type=jax.ShapeDtypeStruct((_NC, n_nodes, d), jnp.float32),
        scratch_types=[
            pltpu.VMEM((nb_c, _EB), jnp.int32),
            pltpu.VMEM((nb_c, _EB), jnp.int32),
            pltpu.VMEM((nb_c, _EB), jnp.float32),
            pltpu.VMEM((_EB, d), jnp.float32),
            pltpu.VMEM((_EB, d), jnp.float32),
            pltpu.VMEM_SHARED((n_nodes, d), jnp.float32),
            pltpu.SemaphoreType.DMA,
            pltpu.SemaphoreType.DMA,
            pltpu.SemaphoreType.DMA,
        ],
    )
    def agg(xt_hbm, src_hbm, dst_hbm, w_hbm, out_hbm,
            src_v, dst_v, w_v, rows0, rows1, acc, sem0, sem1, sem_s):
        cid = lax.axis_index("c")
        sid = lax.axis_index("s")
        wid = cid * _NS + sid

        def zero_row(i, carry):
            for c in range(d // _LANES):
                rows0[i, pl.ds(c * _LANES, _LANES)] = jnp.zeros(
                    (_LANES,), jnp.float32)
            return carry
        lax.fori_loop(0, wr, zero_row, 0)
        r0 = sid * sub_stride
        for ch in range(n_wr):
            pltpu.sync_copy(rows0.at[pl.ds(0, wr)],
                            acc.at[pl.ds(r0 + ch * wr, wr)])
        plsc.subcore_barrier()

        dn = lax.GatherDimensionNumbers(
            offset_dims=(), collapsed_slice_dims=(0,),
            start_index_map=(0,))

        n_streams = 4
        hb = _EB // n_streams

        def gather_start(b, buf, sem):
            for s in range(n_streams):
                pltpu.make_async_copy(
                    xt_hbm.at[src_v.at[b, pl.ds(s * hb, hb)]],
                    buf.at[pl.ds(s * hb, hb)], sem).start()

        def gather_wait(b, buf, sem):
            for s in range(n_streams):
                pltpu.make_async_copy(
                    xt_hbm.at[src_v.at[b, pl.ds(s * hb, hb)]],
                    buf.at[pl.ds(s * hb, hb)], sem).wait()

        def process(b, buf):
            @plsc.parallel_loop(0, _EB, 1, unroll=2)
            def mul_edge(e):
                chunk = w_v[b, pl.ds((e // _LANES) * _LANES, _LANES)]
                wspl = lax.gather(
                    chunk, jnp.full((_LANES, 1), e % _LANES, jnp.int32),
                    dn, (1,), mode=lax.GatherScatterMode.PROMISE_IN_BOUNDS)
                for c in range(d // _LANES):
                    sl = pl.ds(c * _LANES, _LANES)
                    buf[e, sl] = buf[e, sl] * wspl
            pltpu.sync_copy(buf, acc.at[dst_v.at[b]], add=True)

        base = wid * nb
        for ck in range(n_chunks):
            row0 = base + ck * nb_c
            cps = [pltpu.make_async_copy(h.at[pl.ds(row0, nb_c)], v, sem_s)
                   for h, v in ((src_hbm, src_v), (dst_hbm, dst_v),
                                (w_hbm, w_v))]
            for cp in cps:
                cp.start()
            for cp in cps:
                cp.wait()
            gather_start(0, rows0, sem0)
            gather_start(1, rows1, sem1)

            def super_batch(sb, carry):
                b0 = 2 * sb
                gather_wait(b0, rows0, sem0)
                process(b0, rows0)

                @pl.when(sb + 1 < nsb_c)
                def _():
                    gather_start(b0 + 2, rows0, sem0)
                gather_wait(b0 + 1, rows1, sem1)
                process(b0 + 1, rows1)

                @pl.when(sb + 1 < nsb_c)
                def _():
                    gather_start(b0 + 3, rows1, sem1)
                return carry
            lax.fori_loop(0, nsb_c, super_batch, 0)
        plsc.subcore_barrier()

        for ch in range(n_wr):
            rr = r0 + ch * wr
            pltpu.sync_copy(acc.at[pl.ds(rr, wr)],
                            out_hbm.at[cid, pl.ds(rr, wr)])

    return agg


def kernel(x, edge_index, edge_weight, W, b):
    n, d = x.shape
    e = edge_index.shape[1]

    rb = 1000
    grid = n // rb
    xt = pl.pallas_call(
        _linear_body,
        grid=(grid,),
        in_specs=[
            pl.BlockSpec((rb, d), lambda i: (i, 0)),
            pl.BlockSpec((d, d), lambda i: (0, 0)),
            pl.BlockSpec((1, d), lambda i: (0, 0)),
        ],
        out_specs=pl.BlockSpec((rb, d), lambda i: (i, 0)),
        out_shape=jax.ShapeDtypeStruct((n, d), jnp.float32),
    )(x, W, b.reshape(1, d))

    tile_chunk = _NC * _NS * _EB * 2
    e_pad = ((e + tile_chunk - 1) // tile_chunk) * tile_chunk
    pad = e_pad - e
    src = jnp.pad(edge_index[0].astype(jnp.int32), (0, pad)).reshape(-1, _EB)
    dst = jnp.pad(edge_index[1].astype(jnp.int32), (0, pad)).reshape(-1, _EB)
    w_e = jnp.pad(edge_weight, (0, pad)).reshape(-1, _EB)
    partial = _make_agg(n, d, e_pad // (_NC * _NS))(xt, src, dst, w_e)

    out = pl.pallas_call(
        _epilogue_body,
        grid=(grid,),
        in_specs=[
            pl.BlockSpec((rb, d), lambda i: (i, 0)),
            pl.BlockSpec((rb, d), lambda i: (i, 0)),
        ],
        out_specs=pl.BlockSpec((rb, d), lambda i: (i, 0)),
        out_shape=jax.ShapeDtypeStruct((n, d), jnp.float32),
    )(partial[0], partial[1])
    return out


# speedup vs baseline: 1.0018x; 1.0018x over previous
import functools

import jax
import jax.numpy as jnp
from jax import lax
from jax.experimental import pallas as pl
from jax.experimental.pallas import tpu as pltpu
from jax.experimental.pallas import tpu_sc as plsc

_MIN_NORM = 1e-15
_LANES = 16
_NC = 2
_NS = 16
_EB = 128


def _artanh(z):
    z = jnp.clip(z, -1.0 + 1e-7, 1.0 - 1e-7)
    return 0.5 * (jnp.log1p(z) - jnp.log1p(-z))


def _rnorm(v):
    return jnp.maximum(jnp.sqrt(jnp.sum(v * v, axis=-1, keepdims=True)),
                       _MIN_NORM)


def _proj(v):
    n = _rnorm(v)
    maxn = 1.0 - 1e-5
    return jnp.where(n > maxn, v / n * maxn, v)


def _expmap0(u):
    n = _rnorm(u)
    return jnp.tanh(n) * u / n


def _logmap0(p):
    n = _rnorm(p)
    return p / n * _artanh(n)


def _mobius_add(x, y):
    x2 = jnp.sum(x * x, axis=-1, keepdims=True)
    y2 = jnp.sum(y * y, axis=-1, keepdims=True)
    xy = jnp.sum(x * y, axis=-1, keepdims=True)
    num = (1.0 + 2.0 * xy + y2) * x + (1.0 - x2) * y
    den = 1.0 + 2.0 * xy + x2 * y2
    return num / jnp.maximum(den, _MIN_NORM)


def _linear_body(x_ref, w_ref, b_ref, o_ref):
    xb = x_ref[...]
    w = w_ref[...]
    b = b_ref[...]
    mx = lax.dot_general(xb, w, (((1,), (1,)), ((), ())),
                         preferred_element_type=jnp.float32)
    x_n = _rnorm(xb)
    mx_n = _rnorm(mx)
    mv = jnp.tanh(mx_n / x_n * _artanh(x_n)) * mx / mx_n
    res = _proj(mv)
    hb = _proj(_expmap0(b))
    res = _proj(_mobius_add(res, hb))
    o_ref[...] = _logmap0(res)


def _epilogue_body(a_ref, b_ref, o_ref):
    s = a_ref[...] + b_ref[...]
    agg = _proj(_expmap0(s))
    xt = jnp.maximum(_logmap0(agg), 0.0)
    o_ref[...] = _proj(_expmap0(xt))


def _make_agg(n_nodes, d, e_per_tile):
    nb = e_per_tile // _EB
    assert nb % 2 == 0
    nb_c = 8
    if nb % 16 == 0:
        nb_c = 16
    assert nb % nb_c == 0
    n_chunks = nb // nb_c
    nsb_c = nb_c // 2
    sub_stride = 624
    n_wr = 5
    wr = _EB
    assert (_NS - 1) * sub_stride + n_wr * wr == n_nodes
    mesh = plsc.VectorSubcoreMesh(core_axis_name="c", subcore_axis_name="s")

    @functools.partial(
        pl.kernel,
        mesh=mesh,
        out_type=jax.ShapeDtypeStruct((_NC, n_nodes, d), jnp.float32),
        scratch_types=[
            pltpu.VMEM((nb_c, _EB), jnp.int32),
            pltpu.VMEM((nb_c, _EB), jnp.int32),
            pltpu.VMEM((nb_c, _EB), jnp.float32),
            pltpu.VMEM((_EB, d), jnp.float32),
            pltpu.VMEM((_EB, d), jnp.float32),
            pltpu.VMEM_SHARED((n_nodes, d), jnp.float32),
            pltpu.SemaphoreType.DMA,
            pltpu.SemaphoreType.DMA,
            pltpu.SemaphoreType.DMA,
        ],
    )
    def agg(xt_hbm, src_hbm, dst_hbm, w_hbm, out_hbm,
            src_v, dst_v, w_v, rows0, rows1, acc, sem0, sem1, sem_s):
        cid = lax.axis_index("c")
        sid = lax.axis_index("s")
        wid = cid * _NS + sid

        def zero_row(i, carry):
            for c in range(d // _LANES):
                rows0[i, pl.ds(c * _LANES, _LANES)] = jnp.zeros(
                    (_LANES,), jnp.float32)
            return carry
        lax.fori_loop(0, wr, zero_row, 0)
        r0 = sid * sub_stride
        for ch in range(n_wr):
            pltpu.sync_copy(rows0.at[pl.ds(0, wr)],
                            acc.at[pl.ds(r0 + ch * wr, wr)])
        plsc.subcore_barrier()

        dn = lax.GatherDimensionNumbers(
            offset_dims=(), collapsed_slice_dims=(0,),
            start_index_map=(0,))

        n_streams = 2
        hb = _EB // n_streams

        def gather_start(b, buf, sem):
            for s in range(n_streams):
                pltpu.make_async_copy(
                    xt_hbm.at[src_v.at[b, pl.ds(s * hb, hb)]],
                    buf.at[pl.ds(s * hb, hb)], sem).start()

        def gather_wait(b, buf, sem):
            for s in range(n_streams):
                pltpu.make_async_copy(
                    xt_hbm.at[src_v.at[b, pl.ds(s * hb, hb)]],
                    buf.at[pl.ds(s * hb, hb)], sem).wait()

        def process(b, buf):
            @plsc.parallel_loop(0, _EB, 1, unroll=2)
            def mul_edge(e):
                chunk = w_v[b, pl.ds((e // _LANES) * _LANES, _LANES)]
                wspl = lax.gather(
                    chunk, jnp.full((_LANES, 1), e % _LANES, jnp.int32),
                    dn, (1,), mode=lax.GatherScatterMode.PROMISE_IN_BOUNDS)
                for c in range(d // _LANES):
                    sl = pl.ds(c * _LANES, _LANES)
                    buf[e, sl] = buf[e, sl] * wspl
            pltpu.sync_copy(buf, acc.at[dst_v.at[b]], add=True)

        base = wid * nb
        for ck in range(n_chunks):
            row0 = base + ck * nb_c
            cps = [pltpu.make_async_copy(h.at[pl.ds(row0, nb_c)], v, sem_s)
                   for h, v in ((src_hbm, src_v), (dst_hbm, dst_v),
                                (w_hbm, w_v))]
            for cp in cps:
                cp.start()
            for cp in cps:
                cp.wait()
            gather_start(0, rows0, sem0)
            gather_start(1, rows1, sem1)

            def super_batch(sb, carry):
                b0 = 2 * sb
                gather_wait(b0, rows0, sem0)
                process(b0, rows0)

                @pl.when(sb + 1 < nsb_c)
                def _():
                    gather_start(b0 + 2, rows0, sem0)
                gather_wait(b0 + 1, rows1, sem1)
                process(b0 + 1, rows1)

                @pl.when(sb + 1 < nsb_c)
                def _():
                    gather_start(b0 + 3, rows1, sem1)
                return carry
            lax.fori_loop(0, nsb_c, super_batch, 0)
        plsc.subcore_barrier()

        for ch in range(n_wr):
            rr = r0 + ch * wr
            pltpu.sync_copy(acc.at[pl.ds(rr, wr)],
                            out_hbm.at[cid, pl.ds(rr, wr)])

    return agg


def kernel(x, edge_index, edge_weight, W, b):
    n, d = x.shape
    e = edge_index.shape[1]

    rb = 1000
    grid = n // rb
    xt = pl.pallas_call(
        _linear_body,
        grid=(grid,),
        in_specs=[
            pl.BlockSpec((rb, d), lambda i: (i, 0)),
            pl.BlockSpec((d, d), lambda i: (0, 0)),
            pl.BlockSpec((1, d), lambda i: (0, 0)),
        ],
        out_specs=pl.BlockSpec((rb, d), lambda i: (i, 0)),
        out_shape=jax.ShapeDtypeStruct((n, d), jnp.float32),
    )(x, W, b.reshape(1, d))

    tile_chunk = _NC * _NS * _EB * 2
    e_pad = ((e + tile_chunk - 1) // tile_chunk) * tile_chunk
    pad = e_pad - e
    src = jnp.pad(edge_index[0].astype(jnp.int32), (0, pad)).reshape(-1, _EB)
    dst = jnp.pad(edge_index[1].astype(jnp.int32), (0, pad)).reshape(-1, _EB)
    w_e = jnp.pad(edge_weight, (0, pad)).reshape(-1, _EB)
    partial = _make_agg(n, d, e_pad // (_NC * _NS))(xt, src, dst, w_e)

    out = pl.pallas_call(
        _epilogue_body,
        grid=(grid,),
        in_specs=[
            pl.BlockSpec((rb, d), lambda i: (i, 0)),
            pl.BlockSpec((rb, d), lambda i: (i, 0)),
        ],
        out_specs=pl.BlockSpec((rb, d), lambda i: (i, 0)),
        out_shape=jax.ShapeDtypeStruct((n, d), jnp.float32),
    )(partial[0], partial[1])
    return out
